# baseline (speedup 1.0000x reference)
import functools

import jax
import jax.numpy as jnp
from jax import lax
from jax.experimental import pallas as pl
from jax.experimental.pallas import tpu as pltpu
from jax.experimental.pallas import tpu_sc as plsc

DIM = 32
L = 16
CH = 128
NW = 32
BIG = 2 ** 30
BN = 2048


def _relayout_tc(vocab):
    grid = ((vocab + BN - 1) // BN,)

    def body(wt_ref, out_ref):
        t = jnp.transpose(wt_ref[...], (1, 0))
        t3 = t.reshape(BN // 4, 4, DIM)
        for k in range(4):
            out_ref[:, k * DIM:(k + 1) * DIM] = t3[:, k, :]

    return pl.pallas_call(
        body,
        grid=grid,
        in_specs=[pl.BlockSpec((DIM, BN), lambda i: (0, i))],
        out_specs=pl.BlockSpec((BN // 4, 4 * DIM), lambda i: (i, 0)),
        out_shape=jax.ShapeDtypeStruct((vocab // 4, 4 * DIM), jnp.float32),
    )


def _emb_bag_mean(total, b, vocab):
    nbpw = b // NW
    owin = nbpw + 2 * L
    opad = (NW - 1) * nbpw + owin
    nsteps = 1
    while (1 << nsteps) < nbpw + 2:
        nsteps += 1

    mesh = plsc.VectorSubcoreMesh(core_axis_name="c", subcore_axis_name="s")

    @functools.partial(
        pl.kernel,
        out_type=jax.ShapeDtypeStruct((b * DIM,), jnp.float32),
        mesh=mesh,
        scratch_types=[
            pltpu.VMEM((owin,), jnp.int32),
            pltpu.VMEM((CH,), jnp.int32),
            pltpu.VMEM((CH,), jnp.int32),
            pltpu.VMEM((CH + L,), jnp.int32),
            pltpu.VMEM((CH + L,), jnp.int32),
            pltpu.VMEM((CH, 4 * DIM), jnp.float32),
            pltpu.VMEM((CH, 4 * DIM), jnp.float32),
            pltpu.VMEM(((CH + 1) * DIM,), jnp.float32),
            pltpu.VMEM(((nbpw + 1) * DIM,), jnp.float32),
            pltpu.SemaphoreType.DMA,
            pltpu.SemaphoreType.DMA,
            pltpu.SemaphoreType.DMA,
            pltpu.SemaphoreType.DMA,
        ],
    )
    def emb_bag(ids_hbm, offs_hbm, w_hbm, out_hbm, offs_v, ids_v0, ids_v1,
                sub_v0, sub_v1, rows_v0, rows_v1, P_v, S_v, si0, si1, sg0,
                sg1):
        ids_v = (ids_v0, ids_v1)
        sub_v = (sub_v0, sub_v1)
        rows_v = (rows_v0, rows_v1)
        si = (si0, si1)
        sg = (sg0, sg1)

        wid = lax.axis_index("s") * 2 + lax.axis_index("c")
        base = pl.multiple_of(wid * nbpw, nbpw)
        pltpu.sync_copy(offs_hbm.at[pl.ds(base, owin)], offs_v)

        def oread(idx):
            return offs_v[pl.ds(idx, L)][0]

        def ubound(v):
            def step(_, c):
                lo, hi = c
                mid = (lo + hi) // 2
                le = oread(mid) <= v
                return jnp.where(le, mid + 1, lo), jnp.where(le, hi, mid)
            lo, _ = lax.fori_loop(
                0, nsteps, step,
                (jnp.asarray(0, jnp.int32), jnp.asarray(nbpw + 1, jnp.int32)))
            return lo

        start = oread(0)
        end = oread(nbpw)
        start0 = (start // 8) * 8
        zero = jnp.zeros((L,), jnp.float32)

        ub0 = ubound(start0)

        def preset(i, _):
            S_v[pl.ds(i * DIM, L)] = zero
            S_v[pl.ds(i * DIM + L, L)] = zero
            return 0
        lax.fori_loop(0, ub0, preset, 0)

        nch = (end - start0 + CH - 1) // CH
        nch2 = (jnp.maximum(nch, 1) + 1) // 2
        NCH = nch2 * 2
        last = (NCH - 1) * CH

        def addr(j):
            return pl.multiple_of(start0 + jnp.minimum(j * CH, last), 8)

        def prep(par):
            def prep_body(i, _):
                v = ids_v[par][pl.ds(i * L, L)]
                ids_v[par][pl.ds(i * L, L)] = lax.shift_right_logical(v, 2)
                sub_v[par][pl.ds(i * L, L)] = (v & 3) * DIM
                return 0
            lax.fori_loop(0, CH // L, prep_body, 0)

        pltpu.async_copy(ids_hbm.at[pl.ds(addr(0), CH)], ids_v[0],
                         si[0]).wait()
        prep(0)
        pltpu.async_copy(w_hbm.at[ids_v[0]], rows_v[0], sg[0])
        pltpu.async_copy(ids_hbm.at[pl.ds(addr(1), CH)], ids_v[1], si[1])

        def do_chunk(k, par, carry):
            ptr, s0, s1 = carry
            pos = start0 + k * CH
            pltpu.make_async_copy(w_hbm.at[ids_v[par]], rows_v[par],
                                  sg[par]).wait()
            pltpu.make_async_copy(ids_hbm.at[pl.ds(addr(k + 1), CH)],
                                  ids_v[1 - par], si[1 - par]).wait()
            prep(1 - par)
            pltpu.async_copy(w_hbm.at[ids_v[1 - par]], rows_v[1 - par],
                             sg[1 - par])
            pltpu.async_copy(ids_hbm.at[pl.ds(addr(k + 2), CH)], ids_v[par],
                             si[par])

            rows = rows_v[par]
            subs = sub_v[par]

            def scan_body(j, c):
                a0, a1 = c
                o = subs[pl.ds(j, L)][0]
                a0 = a0 + rows[j, pl.ds(o, L)]
                a1 = a1 + rows[j, pl.ds(o + L, L)]
                P_v[pl.ds((j + 1) * DIM, L)] = a0
                P_v[pl.ds((j + 1) * DIM + L, L)] = a1
                return a0, a1
            s0, s1 = lax.fori_loop(0, CH, scan_body, (s0, s1))

            ubk = ubound(pos + CH)

            def ext_body(i, _):
                bi = ptr + i
                rel = oread(bi) - pos
                S_v[pl.ds(bi * DIM, L)] = P_v[pl.ds(rel * DIM, L)]
                S_v[pl.ds(bi * DIM + L, L)] = P_v[pl.ds(rel * DIM + L, L)]
                return 0
            lax.fori_loop(0, ubk - ptr, ext_body, 0)
            return ubk, s0, s1

        def pair_body(k2, carry):
            carry = do_chunk(k2 * 2, 0, carry)
            carry = do_chunk(k2 * 2 + 1, 1, carry)
            return carry

        init = (ub0, zero, zero)
        lax.fori_loop(0, nch2, pair_body, init)

        pltpu.make_async_copy(w_hbm.at[ids_v[0]], rows_v[0], sg[0]).wait()
        pltpu.make_async_copy(ids_hbm.at[pl.ds(addr(NCH + 1), CH)], ids_v[1],
                              si[1]).wait()

        def div_body(i, c):
            p0, p1, bprev = c
            bnext = oread(i + 1)
            cnt = jnp.maximum((bnext - bprev).astype(jnp.float32), 1.0)
            inv = 1.0 / lax.broadcast(cnt, (L,))
            c0 = S_v[pl.ds((i + 1) * DIM, L)]
            c1 = S_v[pl.ds((i + 1) * DIM + L, L)]
            S_v[pl.ds(i * DIM, L)] = (c0 - p0) * inv
            S_v[pl.ds(i * DIM + L, L)] = (c1 - p1) * inv
            return c0, c1, bnext

        p0 = S_v[pl.ds(0, L)]
        p1 = S_v[pl.ds(L, L)]
        lax.fori_loop(0, nbpw, div_body, (p0, p1, start))

        pltpu.sync_copy(S_v.at[pl.ds(0, nbpw * DIM)],
                        out_hbm.at[pl.ds(pl.multiple_of(base * DIM, 8),
                                         nbpw * DIM)])

    return emb_bag


def kernel(ids, offsets, W):
    total = ids.shape[0]
    b = offsets.shape[0]
    vocab = W.shape[0]
    pad_ids = (jnp.arange(2 * CH, dtype=jnp.int32) * 1024) % vocab
    ids_pad = jnp.concatenate([ids, pad_ids])
    nbpw = b // NW
    opad = (NW - 1) * nbpw + nbpw + 2 * L
    offs_ext = jnp.concatenate([
        offsets,
        jnp.asarray([total], jnp.int32),
        jnp.full((opad - b - 1,), BIG, jnp.int32),
    ])
    W4 = _relayout_tc(vocab)(W.T)
    out = _emb_bag_mean(total, b, vocab)(ids_pad, offs_ext, W4)
    return out.reshape(b, DIM)


# speedup vs baseline: 1.6505x; 1.6505x over previous
import functools

import jax
import jax.numpy as jnp
from jax import lax
from jax.experimental import pallas as pl
from jax.experimental.pallas import tpu as pltpu
from jax.experimental.pallas import tpu_sc as plsc

DIM = 32
L = 16
CH = 128
NW = 32
BIG = 2 ** 30
BN = 2048


def _relayout_tc(vocab):
    grid = ((vocab + BN - 1) // BN,)

    def body(wt_ref, out_ref):
        t = jnp.transpose(wt_ref[...], (1, 0))
        t3 = t.reshape(BN // 4, 4, DIM)
        out_ref[...] = jnp.concatenate(
            [t3[:, k, :] for k in range(4)], axis=1)

    return pl.pallas_call(
        body,
        grid=grid,
        in_specs=[pl.BlockSpec((DIM, BN), lambda i: (0, i))],
        out_specs=pl.BlockSpec((BN // 4, 4 * DIM), lambda i: (i, 0)),
        out_shape=jax.ShapeDtypeStruct((vocab // 4, 4 * DIM), jnp.float32),
    )


def _emb_bag_mean(total, b, vocab):
    nbpw = b // NW
    owin = nbpw + 2 * L
    opad = (NW - 1) * nbpw + owin
    nsteps = 1
    while (1 << nsteps) < nbpw + 2:
        nsteps += 1

    mesh = plsc.VectorSubcoreMesh(core_axis_name="c", subcore_axis_name="s")

    @functools.partial(
        pl.kernel,
        out_type=jax.ShapeDtypeStruct((b * DIM,), jnp.float32),
        mesh=mesh,
        compiler_params=pltpu.CompilerParams(use_tc_tiling_on_sc=False),
        scratch_types=[
            pltpu.VMEM((owin,), jnp.int32),
            pltpu.VMEM((CH,), jnp.int32),
            pltpu.VMEM((CH,), jnp.int32),
            pltpu.VMEM((CH, DIM), jnp.float32),
            pltpu.VMEM((CH, DIM), jnp.float32),
            pltpu.VMEM(((CH + 1) * DIM,), jnp.float32),
            pltpu.VMEM(((nbpw + 1) * DIM,), jnp.float32),
            pltpu.SemaphoreType.DMA,
            pltpu.SemaphoreType.DMA,
            pltpu.SemaphoreType.DMA,
            pltpu.SemaphoreType.DMA,
        ],
    )
    def emb_bag(ids_hbm, offs_hbm, w_hbm, out_hbm, offs_v, ids_v0, ids_v1,
                rows_v0, rows_v1, P_v, S_v, si0, si1, sg0, sg1):
        ids_v = (ids_v0, ids_v1)
        rows_v = (rows_v0, rows_v1)
        si = (si0, si1)
        sg = (sg0, sg1)

        wid = lax.axis_index("s") * 2 + lax.axis_index("c")
        base = pl.multiple_of(wid * nbpw, nbpw)
        pltpu.sync_copy(offs_hbm.at[pl.ds(base, owin)], offs_v)

        def oread(idx):
            return offs_v[pl.ds(idx, L)][0]

        def ubound(v):
            def step(_, c):
                lo, hi = c
                mid = (lo + hi) // 2
                le = oread(mid) <= v
                return jnp.where(le, mid + 1, lo), jnp.where(le, hi, mid)
            lo, _ = lax.fori_loop(
                0, nsteps, step,
                (jnp.asarray(0, jnp.int32), jnp.asarray(nbpw + 1, jnp.int32)),
                unroll=True)
            return lo

        start = oread(0)
        end = oread(nbpw)
        start0 = (start // 8) * 8
        zero = jnp.zeros((L,), jnp.float32)

        ub0 = ubound(start0)

        def preset(i, _):
            S_v[pl.ds(i * DIM, L)] = zero
            S_v[pl.ds(i * DIM + L, L)] = zero
            return 0
        lax.fori_loop(0, ub0, preset, 0)

        nch = (end - start0 + CH - 1) // CH
        nch2 = (jnp.maximum(nch, 1) + 1) // 2
        NCH = nch2 * 2
        last = (NCH - 1) * CH

        def addr(j):
            return pl.multiple_of(start0 + jnp.minimum(j * CH, last), 8)

        pltpu.async_copy(ids_hbm.at[pl.ds(addr(0), CH)], ids_v[0],
                         si[0]).wait()
        pltpu.async_copy(w_hbm.at[ids_v[0]], rows_v[0], sg[0])
        pltpu.async_copy(ids_hbm.at[pl.ds(addr(1), CH)], ids_v[1], si[1])

        def do_chunk(k, par, carry):
            ptr, s0, s1 = carry
            pos = start0 + k * CH
            pltpu.make_async_copy(w_hbm.at[ids_v[par]], rows_v[par],
                                  sg[par]).wait()
            pltpu.make_async_copy(ids_hbm.at[pl.ds(addr(k + 1), CH)],
                                  ids_v[1 - par], si[1 - par]).wait()
            pltpu.async_copy(w_hbm.at[ids_v[1 - par]], rows_v[1 - par],
                             sg[1 - par])
            pltpu.async_copy(ids_hbm.at[pl.ds(addr(k + 2), CH)], ids_v[par],
                             si[par])

            rows = rows_v[par]

            def scan_body(j, c):
                a0, a1 = c
                a0 = a0 + rows[j, pl.ds(0, L)]
                a1 = a1 + rows[j, pl.ds(L, L)]
                P_v[pl.ds((j + 1) * DIM, L)] = a0
                P_v[pl.ds((j + 1) * DIM + L, L)] = a1
                return a0, a1
            s0, s1 = lax.fori_loop(0, CH, scan_body, (s0, s1), unroll=8)

            ubk = ubound(pos + CH)

            def ext_body(i, _):
                bi = ptr + i
                rel = oread(bi) - pos
                S_v[pl.ds(bi * DIM, L)] = P_v[pl.ds(rel * DIM, L)]
                S_v[pl.ds(bi * DIM + L, L)] = P_v[pl.ds(rel * DIM + L, L)]
                return 0
            lax.fori_loop(0, ubk - ptr, ext_body, 0)
            return ubk, s0, s1

        def pair_body(k2, carry):
            carry = do_chunk(k2 * 2, 0, carry)
            carry = do_chunk(k2 * 2 + 1, 1, carry)
            return carry

        init = (ub0, zero, zero)
        lax.fori_loop(0, nch2, pair_body, init)

        pltpu.make_async_copy(w_hbm.at[ids_v[0]], rows_v[0], sg[0]).wait()
        pltpu.make_async_copy(ids_hbm.at[pl.ds(addr(NCH + 1), CH)], ids_v[1],
                              si[1]).wait()

        def div_body(i, c):
            p0, p1, bprev = c
            bnext = oread(i + 1)
            cnt = jnp.maximum((bnext - bprev).astype(jnp.float32), 1.0)
            inv = 1.0 / lax.broadcast(cnt, (L,))
            c0 = S_v[pl.ds((i + 1) * DIM, L)]
            c1 = S_v[pl.ds((i + 1) * DIM + L, L)]
            S_v[pl.ds(i * DIM, L)] = (c0 - p0) * inv
            S_v[pl.ds(i * DIM + L, L)] = (c1 - p1) * inv
            return c0, c1, bnext

        p0 = S_v[pl.ds(0, L)]
        p1 = S_v[pl.ds(L, L)]
        lax.fori_loop(0, nbpw, div_body, (p0, p1, start))

        pltpu.sync_copy(S_v.at[pl.ds(0, nbpw * DIM)],
                        out_hbm.at[pl.ds(pl.multiple_of(base * DIM, 8),
                                         nbpw * DIM)])

    return emb_bag


def kernel(ids, offsets, W):
    total = ids.shape[0]
    b = offsets.shape[0]
    vocab = W.shape[0]
    pad_ids = (jnp.arange(2 * CH, dtype=jnp.int32) * 1024) % vocab
    ids_pad = jnp.concatenate([ids, pad_ids])
    nbpw = b // NW
    opad = (NW - 1) * nbpw + nbpw + 2 * L
    offs_ext = jnp.concatenate([
        offsets,
        jnp.asarray([total], jnp.int32),
        jnp.full((opad - b - 1,), BIG, jnp.int32),
    ])
    W4 = _relayout_tc(vocab)(W.T)
    W_lin = W4.reshape(vocab, DIM)
    out = _emb_bag_mean(total, b, vocab)(ids_pad, offs_ext, W_lin)
    return out.reshape(b, DIM)


# speedup vs baseline: 1.9280x; 1.1682x over previous
import functools

import jax
import jax.numpy as jnp
from jax import lax
from jax.experimental import pallas as pl
from jax.experimental.pallas import tpu as pltpu
from jax.experimental.pallas import tpu_sc as plsc

DIM = 32
L = 16
CH = 128
NW = 32
BIG = 2 ** 30
BN = 4096


def _relayout_tc(vocab):
    grid = ((vocab + BN - 1) // BN,)

    def body(wt_ref, out_ref):
        t = jnp.transpose(wt_ref[...], (1, 0))
        t3 = t.reshape(BN // 4, 4, DIM)
        out_ref[...] = jnp.concatenate(
            [t3[:, k, :] for k in range(4)], axis=1)

    return pl.pallas_call(
        body,
        grid=grid,
        in_specs=[pl.BlockSpec((DIM, BN), lambda i: (0, i))],
        out_specs=pl.BlockSpec((BN // 4, 4 * DIM), lambda i: (i, 0)),
        out_shape=jax.ShapeDtypeStruct((vocab // 4, 4 * DIM), jnp.float32),
    )


def _emb_bag_mean(total, b, vocab):
    nbpw = b // NW
    owin = nbpw + 2 * L
    opad = (NW - 1) * nbpw + owin
    nsteps = 1
    while (1 << nsteps) < nbpw + 2:
        nsteps += 1

    mesh = plsc.VectorSubcoreMesh(core_axis_name="c", subcore_axis_name="s")

    @functools.partial(
        pl.kernel,
        out_type=jax.ShapeDtypeStruct((b * DIM,), jnp.float32),
        mesh=mesh,
        compiler_params=pltpu.CompilerParams(use_tc_tiling_on_sc=False),
        scratch_types=[
            pltpu.VMEM((owin,), jnp.int32),
            pltpu.VMEM((CH,), jnp.int32),
            pltpu.VMEM((CH,), jnp.int32),
            pltpu.VMEM((CH, DIM), jnp.float32),
            pltpu.VMEM((CH, DIM), jnp.float32),
            pltpu.VMEM(((CH + 1) * DIM,), jnp.float32),
            pltpu.VMEM(((nbpw + 1) * DIM,), jnp.float32),
            pltpu.SemaphoreType.DMA,
            pltpu.SemaphoreType.DMA,
            pltpu.SemaphoreType.DMA,
            pltpu.SemaphoreType.DMA,
        ],
    )
    def emb_bag(ids_hbm, offs_hbm, w_hbm, out_hbm, offs_v, ids_v0, ids_v1,
                rows_v0, rows_v1, P_v, S_v, si0, si1, sg0, sg1):
        ids_v = (ids_v0, ids_v1)
        rows_v = (rows_v0, rows_v1)
        si = (si0, si1)
        sg = (sg0, sg1)

        wid = lax.axis_index("s") * 2 + lax.axis_index("c")
        base = pl.multiple_of(wid * nbpw, nbpw)
        pltpu.sync_copy(offs_hbm.at[pl.ds(base, owin)], offs_v)

        def oread(idx):
            return offs_v[pl.ds(idx, L)][0]

        def ubound(v):
            def step(_, c):
                lo, hi = c
                mid = (lo + hi) // 2
                le = oread(mid) <= v
                return jnp.where(le, mid + 1, lo), jnp.where(le, hi, mid)
            lo, _ = lax.fori_loop(
                0, nsteps, step,
                (jnp.asarray(0, jnp.int32), jnp.asarray(nbpw + 1, jnp.int32)))
            return lo

        start = oread(0)
        end = oread(nbpw)
        start0 = (start // 8) * 8
        zero = jnp.zeros((L,), jnp.float32)

        ub0 = ubound(start0)

        def preset(i, _):
            S_v[pl.ds(i * DIM, L)] = zero
            S_v[pl.ds(i * DIM + L, L)] = zero
            return 0
        lax.fori_loop(0, ub0, preset, 0)

        nch = (end - start0 + CH - 1) // CH
        nch2 = (jnp.maximum(nch, 1) + 1) // 2
        NCH = nch2 * 2
        last = (NCH - 1) * CH

        def addr(j):
            return pl.multiple_of(start0 + jnp.minimum(j * CH, last), 8)

        pltpu.async_copy(ids_hbm.at[pl.ds(addr(0), CH)], ids_v[0],
                         si[0]).wait()
        pltpu.async_copy(w_hbm.at[ids_v[0]], rows_v[0], sg[0])
        pltpu.async_copy(ids_hbm.at[pl.ds(addr(1), CH)], ids_v[1], si[1])

        def do_chunk(k, par, carry):
            ptr, s0, s1 = carry
            pos = start0 + k * CH
            pltpu.make_async_copy(w_hbm.at[ids_v[par]], rows_v[par],
                                  sg[par]).wait()
            pltpu.make_async_copy(ids_hbm.at[pl.ds(addr(k + 1), CH)],
                                  ids_v[1 - par], si[1 - par]).wait()
            pltpu.async_copy(w_hbm.at[ids_v[1 - par]], rows_v[1 - par],
                             sg[1 - par])
            pltpu.async_copy(ids_hbm.at[pl.ds(addr(k + 2), CH)], ids_v[par],
                             si[par])

            rows = rows_v[par]

            def scan_body(j, c):
                a0, a1 = c
                a0 = a0 + rows[j, pl.ds(0, L)]
                a1 = a1 + rows[j, pl.ds(L, L)]
                P_v[pl.ds((j + 1) * DIM, L)] = a0
                P_v[pl.ds((j + 1) * DIM + L, L)] = a1
                return a0, a1
            s0, s1 = lax.fori_loop(0, CH, scan_body, (s0, s1))

            ubk = ubound(pos + CH)

            def ext_body(i, _):
                bi = ptr + i
                rel = oread(bi) - pos
                S_v[pl.ds(bi * DIM, L)] = P_v[pl.ds(rel * DIM, L)]
                S_v[pl.ds(bi * DIM + L, L)] = P_v[pl.ds(rel * DIM + L, L)]
                return 0
            lax.fori_loop(0, ubk - ptr, ext_body, 0)
            return ubk, s0, s1

        def pair_body(k2, carry):
            carry = do_chunk(k2 * 2, 0, carry)
            carry = do_chunk(k2 * 2 + 1, 1, carry)
            return carry

        init = (ub0, zero, zero)
        lax.fori_loop(0, nch2, pair_body, init)

        pltpu.make_async_copy(w_hbm.at[ids_v[0]], rows_v[0], sg[0]).wait()
        pltpu.make_async_copy(ids_hbm.at[pl.ds(addr(NCH + 1), CH)], ids_v[1],
                              si[1]).wait()

        def div_body(i, c):
            p0, p1, bprev = c
            bnext = oread(i + 1)
            cnt = jnp.maximum((bnext - bprev).astype(jnp.float32), 1.0)
            inv = 1.0 / lax.broadcast(cnt, (L,))
            c0 = S_v[pl.ds((i + 1) * DIM, L)]
            c1 = S_v[pl.ds((i + 1) * DIM + L, L)]
            S_v[pl.ds(i * DIM, L)] = (c0 - p0) * inv
            S_v[pl.ds(i * DIM + L, L)] = (c1 - p1) * inv
            return c0, c1, bnext

        p0 = S_v[pl.ds(0, L)]
        p1 = S_v[pl.ds(L, L)]
        lax.fori_loop(0, nbpw, div_body, (p0, p1, start))

        pltpu.sync_copy(S_v.at[pl.ds(0, nbpw * DIM)],
                        out_hbm.at[pl.ds(pl.multiple_of(base * DIM, 8),
                                         nbpw * DIM)])

    return emb_bag


def kernel(ids, offsets, W):
    total = ids.shape[0]
    b = offsets.shape[0]
    vocab = W.shape[0]
    pad_ids = (jnp.arange(2 * CH, dtype=jnp.int32) * 1024) % vocab
    ids_pad = jnp.concatenate([ids, pad_ids])
    nbpw = b // NW
    opad = (NW - 1) * nbpw + nbpw + 2 * L
    offs_ext = jnp.concatenate([
        offsets,
        jnp.asarray([total], jnp.int32),
        jnp.full((opad - b - 1,), BIG, jnp.int32),
    ])
    W4 = _relayout_tc(vocab)(W.T)
    W_lin = W4.reshape(vocab, DIM)
    out = _emb_bag_mean(total, b, vocab)(ids_pad, offs_ext, W_lin)
    return out.reshape(b, DIM)


# speedup vs baseline: 1.9547x; 1.0138x over previous
import functools

import jax
import jax.numpy as jnp
from jax import lax
from jax.experimental import pallas as pl
from jax.experimental.pallas import tpu as pltpu
from jax.experimental.pallas import tpu_sc as plsc

DIM = 32
L = 16
CH = 128
NW = 32
BIG = 2 ** 30
BN = 8192


def _relayout_tc(vocab):
    grid = ((vocab + BN - 1) // BN,)

    def body(wt_ref, out_ref):
        t = jnp.transpose(wt_ref[...], (1, 0))
        t3 = t.reshape(BN // 4, 4, DIM)
        out_ref[...] = jnp.concatenate(
            [t3[:, k, :] for k in range(4)], axis=1)

    return pl.pallas_call(
        body,
        grid=grid,
        in_specs=[pl.BlockSpec((DIM, BN), lambda i: (0, i))],
        out_specs=pl.BlockSpec((BN // 4, 4 * DIM), lambda i: (i, 0)),
        out_shape=jax.ShapeDtypeStruct((vocab // 4, 4 * DIM), jnp.float32),
    )


def _emb_bag_mean(total, b, vocab):
    nbpw = b // NW
    owin = nbpw + 2 * L
    opad = (NW - 1) * nbpw + owin
    nsteps = 1
    while (1 << nsteps) < nbpw + 2:
        nsteps += 1

    mesh = plsc.VectorSubcoreMesh(core_axis_name="c", subcore_axis_name="s")

    @functools.partial(
        pl.kernel,
        out_type=jax.ShapeDtypeStruct((b * DIM,), jnp.float32),
        mesh=mesh,
        compiler_params=pltpu.CompilerParams(use_tc_tiling_on_sc=False),
        scratch_types=[
            pltpu.VMEM((owin,), jnp.int32),
            pltpu.VMEM((CH,), jnp.int32),
            pltpu.VMEM((CH,), jnp.int32),
            pltpu.VMEM((CH, DIM), jnp.float32),
            pltpu.VMEM((CH, DIM), jnp.float32),
            pltpu.VMEM(((CH + 1) * DIM,), jnp.float32),
            pltpu.VMEM(((nbpw + 1) * DIM,), jnp.float32),
            pltpu.SemaphoreType.DMA,
            pltpu.SemaphoreType.DMA,
            pltpu.SemaphoreType.DMA,
            pltpu.SemaphoreType.DMA,
        ],
    )
    def emb_bag(ids_hbm, offs_hbm, w_hbm, out_hbm, offs_v, ids_v0, ids_v1,
                rows_v0, rows_v1, P_v, S_v, si0, si1, sg0, sg1):
        ids_v = (ids_v0, ids_v1)
        rows_v = (rows_v0, rows_v1)
        si = (si0, si1)
        sg = (sg0, sg1)

        wid = lax.axis_index("s") * 2 + lax.axis_index("c")
        base = pl.multiple_of(wid * nbpw, nbpw)
        pltpu.sync_copy(offs_hbm.at[pl.ds(base, owin)], offs_v)

        def oread(idx):
            return offs_v[pl.ds(idx, L)][0]

        def ubound(v):
            def step(_, c):
                lo, hi = c
                mid = (lo + hi) // 2
                le = oread(mid) <= v
                return jnp.where(le, mid + 1, lo), jnp.where(le, hi, mid)
            lo, _ = lax.fori_loop(
                0, nsteps, step,
                (jnp.asarray(0, jnp.int32), jnp.asarray(nbpw + 1, jnp.int32)))
            return lo

        start = oread(0)
        end = oread(nbpw)
        start0 = (start // 8) * 8
        zero = jnp.zeros((L,), jnp.float32)

        ub0 = ubound(start0)

        def preset(i, _):
            S_v[pl.ds(i * DIM, L)] = zero
            S_v[pl.ds(i * DIM + L, L)] = zero
            return 0
        lax.fori_loop(0, ub0, preset, 0)

        nch = (end - start0 + CH - 1) // CH
        nch2 = (jnp.maximum(nch, 1) + 1) // 2
        NCH = nch2 * 2
        last = (NCH - 1) * CH

        def addr(j):
            return pl.multiple_of(start0 + jnp.minimum(j * CH, last), 8)

        pltpu.async_copy(ids_hbm.at[pl.ds(addr(0), CH)], ids_v[0],
                         si[0]).wait()
        pltpu.async_copy(w_hbm.at[ids_v[0]], rows_v[0], sg[0])
        pltpu.async_copy(ids_hbm.at[pl.ds(addr(1), CH)], ids_v[1], si[1])

        def do_chunk(k, par, carry):
            ptr, s0, s1 = carry
            pos = start0 + k * CH
            pltpu.make_async_copy(w_hbm.at[ids_v[par]], rows_v[par],
                                  sg[par]).wait()
            pltpu.make_async_copy(ids_hbm.at[pl.ds(addr(k + 1), CH)],
                                  ids_v[1 - par], si[1 - par]).wait()
            pltpu.async_copy(w_hbm.at[ids_v[1 - par]], rows_v[1 - par],
                             sg[1 - par])
            pltpu.async_copy(ids_hbm.at[pl.ds(addr(k + 2), CH)], ids_v[par],
                             si[par])

            rows = rows_v[par]

            def scan_body(j, c):
                a0, a1 = c
                a0 = a0 + rows[j, pl.ds(0, L)]
                a1 = a1 + rows[j, pl.ds(L, L)]
                P_v[pl.ds((j + 1) * DIM, L)] = a0
                P_v[pl.ds((j + 1) * DIM + L, L)] = a1
                return a0, a1
            s0, s1 = lax.fori_loop(0, CH, scan_body, (s0, s1))

            ubk = ubound(pos + CH)

            def ext_body(i, _):
                bi = ptr + i
                rel = oread(bi) - pos
                S_v[pl.ds(bi * DIM, L)] = P_v[pl.ds(rel * DIM, L)]
                S_v[pl.ds(bi * DIM + L, L)] = P_v[pl.ds(rel * DIM + L, L)]
                return 0
            lax.fori_loop(0, ubk - ptr, ext_body, 0)
            return ubk, s0, s1

        def pair_body(k2, carry):
            carry = do_chunk(k2 * 2, 0, carry)
            carry = do_chunk(k2 * 2 + 1, 1, carry)
            return carry

        init = (ub0, zero, zero)
        lax.fori_loop(0, nch2, pair_body, init)

        pltpu.make_async_copy(w_hbm.at[ids_v[0]], rows_v[0], sg[0]).wait()
        pltpu.make_async_copy(ids_hbm.at[pl.ds(addr(NCH + 1), CH)], ids_v[1],
                              si[1]).wait()

        def div_body(i, c):
            p0, p1, bprev = c
            bnext = oread(i + 1)
            cnt = jnp.maximum((bnext - bprev).astype(jnp.float32), 1.0)
            inv = 1.0 / lax.broadcast(cnt, (L,))
            c0 = S_v[pl.ds((i + 1) * DIM, L)]
            c1 = S_v[pl.ds((i + 1) * DIM + L, L)]
            S_v[pl.ds(i * DIM, L)] = (c0 - p0) * inv
            S_v[pl.ds(i * DIM + L, L)] = (c1 - p1) * inv
            return c0, c1, bnext

        p0 = S_v[pl.ds(0, L)]
        p1 = S_v[pl.ds(L, L)]
        lax.fori_loop(0, nbpw, div_body, (p0, p1, start))

        pltpu.sync_copy(S_v.at[pl.ds(0, nbpw * DIM)],
                        out_hbm.at[pl.ds(pl.multiple_of(base * DIM, 8),
                                         nbpw * DIM)])

    return emb_bag


def kernel(ids, offsets, W):
    total = ids.shape[0]
    b = offsets.shape[0]
    vocab = W.shape[0]
    pad_ids = (jnp.arange(2 * CH, dtype=jnp.int32) * 1024) % vocab
    ids_pad = jnp.concatenate([ids, pad_ids])
    nbpw = b // NW
    opad = (NW - 1) * nbpw + nbpw + 2 * L
    offs_ext = jnp.concatenate([
        offsets,
        jnp.asarray([total], jnp.int32),
        jnp.full((opad - b - 1,), BIG, jnp.int32),
    ])
    W4 = _relayout_tc(vocab)(W.T)
    W_lin = W4.reshape(vocab, DIM)
    out = _emb_bag_mean(total, b, vocab)(ids_pad, offs_ext, W_lin)
    return out.reshape(b, DIM)
